# Initial kernel scaffold; baseline (speedup 1.0000x reference)
import jax
import jax.numpy as jnp
from jax.experimental import pallas as pl


def kernel(x, mem, W, b):
    raise NotImplementedError("write your pallas kernel here")


# speedup vs baseline: 1.0252x; 1.0252x over previous
import jax
import jax.numpy as jnp
from jax.experimental import pallas as pl

INPUT_SIZE = 256
OUT_SIZE = 256
MEMORY_FEATURE = 128

_BLOCK_M = 1024


def _matmul_body(x_ref, mem_ref, w_ref, b_ref, out_ref):
    acc = jnp.dot(x_ref[...], w_ref[:INPUT_SIZE, :],
                  preferred_element_type=jnp.float32)
    acc = acc + jnp.dot(mem_ref[...], w_ref[INPUT_SIZE:, :],
                        preferred_element_type=jnp.float32)
    out_ref[...] = acc + b_ref[...]


def kernel(x, mem, W, b):
    batch, _ = x.shape
    nblocks = batch // _BLOCK_M
    b2 = b.reshape(1, OUT_SIZE)
    out = pl.pallas_call(
        _matmul_body,
        grid=(nblocks,),
        in_specs=[
            pl.BlockSpec((_BLOCK_M, INPUT_SIZE), lambda i: (i, 0)),
            pl.BlockSpec((_BLOCK_M, MEMORY_FEATURE), lambda i: (i, 0)),
            pl.BlockSpec((INPUT_SIZE + MEMORY_FEATURE, OUT_SIZE),
                         lambda i: (0, 0)),
            pl.BlockSpec((1, OUT_SIZE), lambda i: (0, 0)),
        ],
        out_specs=pl.BlockSpec((_BLOCK_M, OUT_SIZE), lambda i: (i, 0)),
        out_shape=jax.ShapeDtypeStruct((batch, OUT_SIZE), jnp.float32),
    )(x, mem, W, b2)
    return (out, mem)
